# baseline (speedup 1.0000x reference)
import functools

import jax
import jax.numpy as jnp
from jax.experimental import pallas as pl
from jax.experimental.pallas import tpu as pltpu
from jax.experimental.pallas import tpu_sc as plsc


def _conv2d(x, w, b, stride, pad):
    out = jax.lax.conv_general_dilated(
        x, w, (stride, stride), ((pad, pad), (pad, pad)),
        dimension_numbers=('NCHW', 'OIHW', 'NCHW'))
    return out + b[None, :, None, None]


def _conv_transpose2d(x, w, b, stride, pad):
    k = w.shape[2]
    w_conv = jnp.transpose(jnp.flip(w, (2, 3)), (1, 0, 2, 3))
    p = k - 1 - pad
    out = jax.lax.conv_general_dilated(
        x, w_conv, (1, 1), ((p, p), (p, p)), lhs_dilation=(stride, stride),
        dimension_numbers=('NCHW', 'OIHW', 'NCHW'))
    return out + b[None, :, None, None]


def _vq_body(z_ref, cb_ref, zq_ref):
    z = z_ref[...]
    cb = cb_ref[...]
    z_norm = jnp.sum(z * z, axis=1, keepdims=True)
    cb_norm = jnp.sum(cb * cb, axis=1)[None, :]
    d = (z_norm + cb_norm) - 2.0 * jax.lax.dot_general(
        z, cb, (((1,), (1,)), ((), ())), preferred_element_type=jnp.float32)
    d_min = jnp.min(d, axis=1, keepdims=True)
    k = cb.shape[0]
    iota = jax.lax.broadcasted_iota(jnp.int32, d.shape, 1)
    masked_iota = jnp.where(d == d_min, iota, k)
    zq_ref[...] = jnp.min(masked_iota, axis=1, keepdims=True)


@functools.partial(jax.jit, static_argnames=('blk',))
def _vq_lookup(z_flat, codebook, blk=1024):
    n, d = z_flat.shape
    k = codebook.shape[0]
    grid = n // blk
    return pl.pallas_call(
        _vq_body,
        grid=(grid,),
        in_specs=[
            pl.BlockSpec((blk, d), lambda i: (i, 0)),
            pl.BlockSpec((k, d), lambda i: (0, 0)),
        ],
        out_specs=pl.BlockSpec((blk, 1), lambda i: (i, 0)),
        out_shape=jax.ShapeDtypeStruct((n, 1), jnp.int32),
    )(z_flat, codebook)


def _sc_gather(codebook, idx):
    info = plsc.get_sparse_core_info()
    nw = info.num_cores * info.num_subcores
    n = idx.shape[0]
    d = codebook.shape[1]
    ch = 128
    b_per_w = n // nw
    nch = b_per_w // ch
    cb128 = jnp.pad(codebook, ((0, 0), (0, 128 - d)))
    mesh = plsc.VectorSubcoreMesh(core_axis_name="c", subcore_axis_name="s")

    @functools.partial(
        pl.kernel, mesh=mesh,
        out_type=jax.ShapeDtypeStruct((n, 128), jnp.float32),
        scratch_types=[
            pltpu.VMEM((ch,), jnp.int32),
            pltpu.VMEM((ch, 128), jnp.float32),
            pltpu.SemaphoreType.DMA,
        ],
    )
    def k(cb_hbm, idx_hbm, out_hbm, idx_v, rows_v, sem):
        wid = jax.lax.axis_index("s") * info.num_cores + jax.lax.axis_index("c")
        base = wid * b_per_w
        for ci in range(nch):
            off = base + ci * ch
            pltpu.sync_copy(idx_hbm.at[pl.ds(off, ch)], idx_v)
            pltpu.async_copy(cb_hbm.at[idx_v], rows_v, sem).wait()
            pltpu.sync_copy(rows_v, out_hbm.at[pl.ds(off, ch)])

    return k(cb128, idx)[:, :d]


def _tr_phase_conv(x_nhwc, w, bias, act):
    bsz, hh, ww, c = x_nhwc.shape
    o = w.shape[1]
    xp = jnp.pad(x_nhwc, ((0, 0), (1, 1), (1, 1), (0, 0))).astype(jnp.bfloat16)
    wr = w[:, :, ::-1, ::-1].reshape(c, o, 2, 2, 2, 2)
    wp = wr.transpose(2, 4, 0, 3, 5, 1).reshape(2, 2, c, 4 * o)
    p = jax.lax.conv_general_dilated(
        xp, wp.astype(jnp.bfloat16), (1, 1), 'VALID',
        dimension_numbers=('NHWC', 'HWIO', 'NHWC'),
        preferred_element_type=jnp.float32)
    p = p + jnp.tile(bias, 4)
    if act == 'relu':
        p = jnp.maximum(p, 0.0)
    else:
        p = jax.nn.sigmoid(p)
    p = p.reshape(bsz, hh + 1, ww + 1, 4, o)
    quad = jnp.stack([p[:, 0:hh, 0:ww, 0], p[:, 0:hh, 1:ww + 1, 1],
                      p[:, 1:hh + 1, 0:ww, 2], p[:, 1:hh + 1, 1:ww + 1, 3]],
                     axis=-2)
    return (quad.reshape(bsz, hh, ww, 2, 2, o).transpose(0, 1, 3, 2, 4, 5)
            .reshape(bsz, 2 * hh, 2 * ww, o))


def kernel(x, enc_w1, enc_b1, enc_w2, enc_b2, codebook,
           dec_w1, dec_b1, dec_w2, dec_b2):
    x_nhwc = x.transpose(0, 2, 3, 1)
    h = jax.nn.relu(jax.lax.conv_general_dilated(
        x_nhwc, enc_w1.transpose(2, 3, 1, 0), (2, 2), ((1, 1), (1, 1)),
        dimension_numbers=('NHWC', 'HWIO', 'NHWC')) + enc_b1)
    z_e_nhwc = jax.nn.relu(jax.lax.conv_general_dilated(
        h, enc_w2.transpose(2, 3, 1, 0), (2, 2), ((1, 1), (1, 1)),
        dimension_numbers=('NHWC', 'HWIO', 'NHWC')) + enc_b2)
    z_e = z_e_nhwc.transpose(0, 3, 1, 2)
    z_e_flat = jnp.reshape(z_e, (-1, z_e.shape[1]))
    idx = _vq_lookup(z_e_flat, codebook).reshape(-1)
    z_q = _sc_gather(codebook, idx).reshape(z_e.shape)
    b = x.shape[0]
    h2 = _tr_phase_conv(z_q.transpose(0, 2, 3, 1), dec_w1, dec_b1, 'relu')
    x_rec_nhwc = _tr_phase_conv(h2, dec_w2, dec_b2, 'sigmoid')
    x_recon = x_rec_nhwc.transpose(0, 3, 1, 2)
    return (x_recon, z_q)


# speedup vs baseline: 1.0001x; 1.0001x over previous
import functools

import jax
import jax.numpy as jnp
from jax.experimental import pallas as pl
from jax.experimental.pallas import tpu as pltpu
from jax.experimental.pallas import tpu_sc as plsc


def _conv2d(x, w, b, stride, pad):
    out = jax.lax.conv_general_dilated(
        x, w, (stride, stride), ((pad, pad), (pad, pad)),
        dimension_numbers=('NCHW', 'OIHW', 'NCHW'))
    return out + b[None, :, None, None]


def _conv_transpose2d(x, w, b, stride, pad):
    k = w.shape[2]
    w_conv = jnp.transpose(jnp.flip(w, (2, 3)), (1, 0, 2, 3))
    p = k - 1 - pad
    out = jax.lax.conv_general_dilated(
        x, w_conv, (1, 1), ((p, p), (p, p)), lhs_dilation=(stride, stride),
        dimension_numbers=('NCHW', 'OIHW', 'NCHW'))
    return out + b[None, :, None, None]


def _vq_body(z_ref, cb_ref, zq_ref):
    z = z_ref[...]
    cb = cb_ref[...]
    z_norm = jnp.sum(z * z, axis=1, keepdims=True)
    cb_norm = jnp.sum(cb * cb, axis=1)[None, :]
    d = (z_norm + cb_norm) - 2.0 * jax.lax.dot_general(
        z, cb, (((1,), (1,)), ((), ())), preferred_element_type=jnp.float32)
    d_min = jnp.min(d, axis=1, keepdims=True)
    k = cb.shape[0]
    iota = jax.lax.broadcasted_iota(jnp.int32, d.shape, 1)
    masked_iota = jnp.where(d == d_min, iota, k)
    zq_ref[...] = jnp.min(masked_iota, axis=1, keepdims=True)


@functools.partial(jax.jit, static_argnames=('blk',))
def _vq_lookup(z_flat, codebook, blk=1024):
    n, d = z_flat.shape
    k = codebook.shape[0]
    grid = n // blk
    return pl.pallas_call(
        _vq_body,
        grid=(grid,),
        in_specs=[
            pl.BlockSpec((blk, d), lambda i: (i, 0)),
            pl.BlockSpec((k, d), lambda i: (0, 0)),
        ],
        out_specs=pl.BlockSpec((blk, 1), lambda i: (i, 0)),
        out_shape=jax.ShapeDtypeStruct((n, 1), jnp.int32),
    )(z_flat, codebook)


def _sc_gather(codebook, idx):
    info = plsc.get_sparse_core_info()
    nw = info.num_cores * info.num_subcores
    n = idx.shape[0]
    d = codebook.shape[1]
    ch = 128
    b_per_w = n // nw
    nch = b_per_w // ch
    cb128 = jnp.pad(codebook, ((0, 0), (0, 128 - d)))
    mesh = plsc.VectorSubcoreMesh(core_axis_name="c", subcore_axis_name="s")

    @functools.partial(
        pl.kernel, mesh=mesh,
        out_type=jax.ShapeDtypeStruct((n, 128), jnp.float32),
        scratch_types=[
            pltpu.VMEM((nch, ch), jnp.int32),
            pltpu.VMEM((ch, 128), jnp.float32),
            pltpu.VMEM((ch, 128), jnp.float32),
            pltpu.SemaphoreType.DMA,
            pltpu.SemaphoreType.DMA,
            pltpu.SemaphoreType.DMA,
            pltpu.SemaphoreType.DMA,
        ],
    )
    def k(cb_hbm, idx_hbm, out_hbm, idx_v, rows0, rows1,
          gs0, gs1, os0, os1):
        wid = jax.lax.axis_index("s") * info.num_cores + jax.lax.axis_index("c")
        base = wid * b_per_w
        pltpu.sync_copy(idx_hbm.at[wid], idx_v)
        rows = (rows0, rows1)
        gsem = (gs0, gs1)
        osem = (os0, os1)
        gcp = [pltpu.async_copy(cb_hbm.at[idx_v.at[0]], rows0, gs0), None]
        ocp = [None, None]
        for ci in range(nch):
            bb = ci & 1
            gcp[bb].wait()
            if ci + 1 < nch:
                nb = 1 - bb
                if ocp[nb] is not None:
                    ocp[nb].wait()
                gcp[nb] = pltpu.async_copy(
                    cb_hbm.at[idx_v.at[ci + 1]], rows[nb], gsem[nb])
            ocp[bb] = pltpu.async_copy(
                rows[bb], out_hbm.at[pl.ds(base + ci * ch, ch)], osem[bb])
        ocp[0].wait()
        ocp[1].wait()

    return k(cb128, idx.reshape(nw, nch, ch))[:, :d]


def _tr_phase_conv(x_nhwc, w, bias, act):
    bsz, hh, ww, c = x_nhwc.shape
    o = w.shape[1]
    xp = jnp.pad(x_nhwc, ((0, 0), (1, 1), (1, 1), (0, 0))).astype(jnp.bfloat16)
    wr = w[:, :, ::-1, ::-1].reshape(c, o, 2, 2, 2, 2)
    wp = wr.transpose(2, 4, 0, 3, 5, 1).reshape(2, 2, c, 4 * o)
    p = jax.lax.conv_general_dilated(
        xp, wp.astype(jnp.bfloat16), (1, 1), 'VALID',
        dimension_numbers=('NHWC', 'HWIO', 'NHWC'),
        preferred_element_type=jnp.float32)
    p = p + jnp.tile(bias, 4)
    if act == 'relu':
        p = jnp.maximum(p, 0.0)
    else:
        p = jax.nn.sigmoid(p)
    p = p.reshape(bsz, hh + 1, ww + 1, 4, o)
    quad = jnp.stack([p[:, 0:hh, 0:ww, 0], p[:, 0:hh, 1:ww + 1, 1],
                      p[:, 1:hh + 1, 0:ww, 2], p[:, 1:hh + 1, 1:ww + 1, 3]],
                     axis=-2)
    return (quad.reshape(bsz, hh, ww, 2, 2, o).transpose(0, 1, 3, 2, 4, 5)
            .reshape(bsz, 2 * hh, 2 * ww, o))


def kernel(x, enc_w1, enc_b1, enc_w2, enc_b2, codebook,
           dec_w1, dec_b1, dec_w2, dec_b2):
    x_nhwc = x.transpose(0, 2, 3, 1)
    h = jax.nn.relu(jax.lax.conv_general_dilated(
        x_nhwc, enc_w1.transpose(2, 3, 1, 0), (2, 2), ((1, 1), (1, 1)),
        dimension_numbers=('NHWC', 'HWIO', 'NHWC')) + enc_b1)
    z_e_nhwc = jax.nn.relu(jax.lax.conv_general_dilated(
        h, enc_w2.transpose(2, 3, 1, 0), (2, 2), ((1, 1), (1, 1)),
        dimension_numbers=('NHWC', 'HWIO', 'NHWC')) + enc_b2)
    z_e = z_e_nhwc.transpose(0, 3, 1, 2)
    z_e_flat = jnp.reshape(z_e, (-1, z_e.shape[1]))
    idx = _vq_lookup(z_e_flat, codebook).reshape(-1)
    z_q = _sc_gather(codebook, idx).reshape(z_e.shape)
    b = x.shape[0]
    h2 = _tr_phase_conv(z_q.transpose(0, 2, 3, 1), dec_w1, dec_b1, 'relu')
    x_rec_nhwc = _tr_phase_conv(h2, dec_w2, dec_b2, 'sigmoid')
    x_recon = x_rec_nhwc.transpose(0, 3, 1, 2)
    return (x_recon, z_q)


# speedup vs baseline: 1.1521x; 1.1520x over previous
import functools

import jax
import jax.numpy as jnp
from jax.experimental import pallas as pl
from jax.experimental.pallas import tpu as pltpu
from jax.experimental.pallas import tpu_sc as plsc


def _conv2d(x, w, b, stride, pad):
    out = jax.lax.conv_general_dilated(
        x, w, (stride, stride), ((pad, pad), (pad, pad)),
        dimension_numbers=('NCHW', 'OIHW', 'NCHW'))
    return out + b[None, :, None, None]


def _conv_transpose2d(x, w, b, stride, pad):
    k = w.shape[2]
    w_conv = jnp.transpose(jnp.flip(w, (2, 3)), (1, 0, 2, 3))
    p = k - 1 - pad
    out = jax.lax.conv_general_dilated(
        x, w_conv, (1, 1), ((p, p), (p, p)), lhs_dilation=(stride, stride),
        dimension_numbers=('NCHW', 'OIHW', 'NCHW'))
    return out + b[None, :, None, None]


def _vq_body(z_ref, cb_ref, zq_ref):
    z = z_ref[...]
    cb = cb_ref[...]
    z_norm = jnp.sum(z * z, axis=1, keepdims=True)
    cb_norm = jnp.sum(cb * cb, axis=1)[None, :]
    d = (z_norm + cb_norm) - 2.0 * jax.lax.dot_general(
        z, cb, (((1,), (1,)), ((), ())), preferred_element_type=jnp.float32)
    d_min = jnp.min(d, axis=1, keepdims=True)
    k = cb.shape[0]
    iota = jax.lax.broadcasted_iota(jnp.int32, d.shape, 1)
    masked_iota = jnp.where(d == d_min, iota, k)
    zq_ref[...] = jnp.min(masked_iota, axis=1, keepdims=True)


@functools.partial(jax.jit, static_argnames=('blk',))
def _vq_lookup(z_flat, codebook, blk=1024):
    n, d = z_flat.shape
    k = codebook.shape[0]
    grid = n // blk
    return pl.pallas_call(
        _vq_body,
        grid=(grid,),
        in_specs=[
            pl.BlockSpec((blk, d), lambda i: (i, 0)),
            pl.BlockSpec((k, d), lambda i: (0, 0)),
        ],
        out_specs=pl.BlockSpec((blk, 1), lambda i: (i, 0)),
        out_shape=jax.ShapeDtypeStruct((n, 1), jnp.int32),
    )(z_flat, codebook)


def _sc_gather(codebook, idx):
    info = plsc.get_sparse_core_info()
    nw = info.num_cores * info.num_subcores
    n = idx.shape[0]
    kk, d = codebook.shape
    ch = 128
    b_per_w = n // nw
    nch = b_per_w // ch
    nrep = 8
    cb128 = jnp.tile(jnp.pad(codebook, ((0, 0), (0, 128 - d))), (nrep, 1))
    mesh = plsc.VectorSubcoreMesh(core_axis_name="c", subcore_axis_name="s")

    @functools.partial(
        pl.kernel, mesh=mesh,
        out_type=jax.ShapeDtypeStruct((n, 128), jnp.float32),
        scratch_types=[
            pltpu.VMEM((b_per_w,), jnp.int32),
            pltpu.VMEM((ch, 128), jnp.float32),
            pltpu.VMEM((ch, 128), jnp.float32),
            pltpu.SemaphoreType.DMA,
            pltpu.SemaphoreType.DMA,
            pltpu.SemaphoreType.DMA,
            pltpu.SemaphoreType.DMA,
        ],
    )
    def k(cb_hbm, idx_hbm, out_hbm, idx_v, rows0, rows1,
          gs0, gs1, os0, os1):
        wid = jax.lax.axis_index("s") * info.num_cores + jax.lax.axis_index("c")
        base = wid * b_per_w
        pltpu.sync_copy(idx_hbm.at[pl.ds(base, b_per_w)], idx_v)
        rep_off = jax.lax.rem(wid, nrep) * kk
        for q in range(b_per_w // 16):
            idx_v[pl.ds(q * 16, 16)] = idx_v[pl.ds(q * 16, 16)] + rep_off
        rows = (rows0, rows1)
        gsem = (gs0, gs1)
        osem = (os0, os1)
        gcp = [pltpu.async_copy(
            cb_hbm.at[idx_v.at[pl.ds(0, ch)]], rows0, gs0), None]
        ocp = [None, None]
        for ci in range(nch):
            bb = ci & 1
            gcp[bb].wait()
            if ci + 1 < nch:
                nb = 1 - bb
                if ocp[nb] is not None:
                    ocp[nb].wait()
                gcp[nb] = pltpu.async_copy(
                    cb_hbm.at[idx_v.at[pl.ds((ci + 1) * ch, ch)]],
                    rows[nb], gsem[nb])
            ocp[bb] = pltpu.async_copy(
                rows[bb], out_hbm.at[pl.ds(base + ci * ch, ch)], osem[bb])
        ocp[0].wait()
        ocp[1].wait()

    return k(cb128, idx)[:, :d]


def _tr_phase_conv(x_nhwc, w, bias, act):
    bsz, hh, ww, c = x_nhwc.shape
    o = w.shape[1]
    xp = jnp.pad(x_nhwc, ((0, 0), (1, 1), (1, 1), (0, 0))).astype(jnp.bfloat16)
    wr = w[:, :, ::-1, ::-1].reshape(c, o, 2, 2, 2, 2)
    wp = wr.transpose(2, 4, 0, 3, 5, 1).reshape(2, 2, c, 4 * o)
    p = jax.lax.conv_general_dilated(
        xp, wp.astype(jnp.bfloat16), (1, 1), 'VALID',
        dimension_numbers=('NHWC', 'HWIO', 'NHWC'),
        preferred_element_type=jnp.float32)
    p = p + jnp.tile(bias, 4)
    if act == 'relu':
        p = jnp.maximum(p, 0.0)
    else:
        p = jax.nn.sigmoid(p)
    p = p.reshape(bsz, hh + 1, ww + 1, 4, o)
    quad = jnp.stack([p[:, 0:hh, 0:ww, 0], p[:, 0:hh, 1:ww + 1, 1],
                      p[:, 1:hh + 1, 0:ww, 2], p[:, 1:hh + 1, 1:ww + 1, 3]],
                     axis=-2)
    return (quad.reshape(bsz, hh, ww, 2, 2, o).transpose(0, 1, 3, 2, 4, 5)
            .reshape(bsz, 2 * hh, 2 * ww, o))


def kernel(x, enc_w1, enc_b1, enc_w2, enc_b2, codebook,
           dec_w1, dec_b1, dec_w2, dec_b2):
    x_nhwc = x.transpose(0, 2, 3, 1)
    h = jax.nn.relu(jax.lax.conv_general_dilated(
        x_nhwc, enc_w1.transpose(2, 3, 1, 0), (2, 2), ((1, 1), (1, 1)),
        dimension_numbers=('NHWC', 'HWIO', 'NHWC')) + enc_b1)
    z_e_nhwc = jax.nn.relu(jax.lax.conv_general_dilated(
        h, enc_w2.transpose(2, 3, 1, 0), (2, 2), ((1, 1), (1, 1)),
        dimension_numbers=('NHWC', 'HWIO', 'NHWC')) + enc_b2)
    z_e = z_e_nhwc.transpose(0, 3, 1, 2)
    z_e_flat = jnp.reshape(z_e, (-1, z_e.shape[1]))
    idx = _vq_lookup(z_e_flat, codebook).reshape(-1)
    z_q = _sc_gather(codebook, idx).reshape(z_e.shape)
    b = x.shape[0]
    h2 = _tr_phase_conv(z_q.transpose(0, 2, 3, 1), dec_w1, dec_b1, 'relu')
    x_rec_nhwc = _tr_phase_conv(h2, dec_w2, dec_b2, 'sigmoid')
    x_recon = x_rec_nhwc.transpose(0, 3, 1, 2)
    return (x_recon, z_q)


# speedup vs baseline: 1.1851x; 1.0287x over previous
import functools

import jax
import jax.numpy as jnp
from jax.experimental import pallas as pl
from jax.experimental.pallas import tpu as pltpu
from jax.experimental.pallas import tpu_sc as plsc


def _conv2d(x, w, b, stride, pad):
    out = jax.lax.conv_general_dilated(
        x, w, (stride, stride), ((pad, pad), (pad, pad)),
        dimension_numbers=('NCHW', 'OIHW', 'NCHW'))
    return out + b[None, :, None, None]


def _conv_transpose2d(x, w, b, stride, pad):
    k = w.shape[2]
    w_conv = jnp.transpose(jnp.flip(w, (2, 3)), (1, 0, 2, 3))
    p = k - 1 - pad
    out = jax.lax.conv_general_dilated(
        x, w_conv, (1, 1), ((p, p), (p, p)), lhs_dilation=(stride, stride),
        dimension_numbers=('NCHW', 'OIHW', 'NCHW'))
    return out + b[None, :, None, None]


def _vq_body(z_ref, cb_ref, zq_ref):
    z = z_ref[...]
    cb = cb_ref[...]
    z_norm = jnp.sum(z * z, axis=1, keepdims=True)
    cb_norm = jnp.sum(cb * cb, axis=1)[None, :]
    d = (z_norm + cb_norm) - 2.0 * jax.lax.dot_general(
        z, cb, (((1,), (1,)), ((), ())), preferred_element_type=jnp.float32)
    d_min = jnp.min(d, axis=1, keepdims=True)
    k = cb.shape[0]
    iota = jax.lax.broadcasted_iota(jnp.int32, d.shape, 1)
    masked_iota = jnp.where(d == d_min, iota, k)
    zq_ref[...] = jnp.min(masked_iota, axis=1, keepdims=True)


@functools.partial(jax.jit, static_argnames=('blk',))
def _vq_lookup(z_flat, codebook, blk=1024):
    n, d = z_flat.shape
    k = codebook.shape[0]
    grid = n // blk
    return pl.pallas_call(
        _vq_body,
        grid=(grid,),
        in_specs=[
            pl.BlockSpec((blk, d), lambda i: (i, 0)),
            pl.BlockSpec((k, d), lambda i: (0, 0)),
        ],
        out_specs=pl.BlockSpec((blk, 1), lambda i: (i, 0)),
        out_shape=jax.ShapeDtypeStruct((n, 1), jnp.int32),
    )(z_flat, codebook)


def _sc_gather(codebook, idx):
    info = plsc.get_sparse_core_info()
    nw = info.num_cores * info.num_subcores
    n = idx.shape[0]
    kk, d = codebook.shape
    ch = 128
    b_per_w = n // nw
    nch = b_per_w // ch
    nrep = 32
    cb128 = jnp.tile(jnp.pad(codebook, ((0, 0), (0, 128 - d))), (nrep, 1))
    mesh = plsc.VectorSubcoreMesh(core_axis_name="c", subcore_axis_name="s")

    @functools.partial(
        pl.kernel, mesh=mesh,
        out_type=jax.ShapeDtypeStruct((n, 128), jnp.float32),
        scratch_types=[
            pltpu.VMEM((b_per_w,), jnp.int32),
            pltpu.VMEM((ch, 128), jnp.float32),
            pltpu.VMEM((ch, 128), jnp.float32),
            pltpu.SemaphoreType.DMA,
            pltpu.SemaphoreType.DMA,
            pltpu.SemaphoreType.DMA,
            pltpu.SemaphoreType.DMA,
        ],
    )
    def k(cb_hbm, idx_hbm, out_hbm, idx_v, rows0, rows1,
          gs0, gs1, os0, os1):
        wid = jax.lax.axis_index("s") * info.num_cores + jax.lax.axis_index("c")
        base = wid * b_per_w
        pltpu.sync_copy(idx_hbm.at[pl.ds(base, b_per_w)], idx_v)
        rep_off = jax.lax.rem(wid, nrep) * kk
        for q in range(b_per_w // 16):
            idx_v[pl.ds(q * 16, 16)] = idx_v[pl.ds(q * 16, 16)] + rep_off
        rows = (rows0, rows1)
        gsem = (gs0, gs1)
        osem = (os0, os1)
        gcp = [pltpu.async_copy(
            cb_hbm.at[idx_v.at[pl.ds(0, ch)]], rows0, gs0), None]
        ocp = [None, None]
        for ci in range(nch):
            bb = ci & 1
            gcp[bb].wait()
            if ci + 1 < nch:
                nb = 1 - bb
                if ocp[nb] is not None:
                    ocp[nb].wait()
                gcp[nb] = pltpu.async_copy(
                    cb_hbm.at[idx_v.at[pl.ds((ci + 1) * ch, ch)]],
                    rows[nb], gsem[nb])
            ocp[bb] = pltpu.async_copy(
                rows[bb], out_hbm.at[pl.ds(base + ci * ch, ch)], osem[bb])
        ocp[0].wait()
        ocp[1].wait()

    return k(cb128, idx)[:, :d]


def _tr_phase_conv(x_nhwc, w, bias, act):
    bsz, hh, ww, c = x_nhwc.shape
    o = w.shape[1]
    xp = jnp.pad(x_nhwc, ((0, 0), (1, 1), (1, 1), (0, 0))).astype(jnp.bfloat16)
    wr = w[:, :, ::-1, ::-1].reshape(c, o, 2, 2, 2, 2)
    wp = wr.transpose(2, 4, 0, 3, 5, 1).reshape(2, 2, c, 4 * o)
    p = jax.lax.conv_general_dilated(
        xp, wp.astype(jnp.bfloat16), (1, 1), 'VALID',
        dimension_numbers=('NHWC', 'HWIO', 'NHWC'),
        preferred_element_type=jnp.float32)
    p = p + jnp.tile(bias, 4)
    if act == 'relu':
        p = jnp.maximum(p, 0.0)
    else:
        p = jax.nn.sigmoid(p)
    p = p.reshape(bsz, hh + 1, ww + 1, 4, o)
    quad = jnp.stack([p[:, 0:hh, 0:ww, 0], p[:, 0:hh, 1:ww + 1, 1],
                      p[:, 1:hh + 1, 0:ww, 2], p[:, 1:hh + 1, 1:ww + 1, 3]],
                     axis=-2)
    return (quad.reshape(bsz, hh, ww, 2, 2, o).transpose(0, 1, 3, 2, 4, 5)
            .reshape(bsz, 2 * hh, 2 * ww, o))


def kernel(x, enc_w1, enc_b1, enc_w2, enc_b2, codebook,
           dec_w1, dec_b1, dec_w2, dec_b2):
    x_nhwc = x.transpose(0, 2, 3, 1)
    h = jax.nn.relu(jax.lax.conv_general_dilated(
        x_nhwc, enc_w1.transpose(2, 3, 1, 0), (2, 2), ((1, 1), (1, 1)),
        dimension_numbers=('NHWC', 'HWIO', 'NHWC')) + enc_b1)
    z_e_nhwc = jax.nn.relu(jax.lax.conv_general_dilated(
        h, enc_w2.transpose(2, 3, 1, 0), (2, 2), ((1, 1), (1, 1)),
        dimension_numbers=('NHWC', 'HWIO', 'NHWC')) + enc_b2)
    z_e = z_e_nhwc.transpose(0, 3, 1, 2)
    z_e_flat = jnp.reshape(z_e, (-1, z_e.shape[1]))
    idx = _vq_lookup(z_e_flat, codebook).reshape(-1)
    z_q = _sc_gather(codebook, idx).reshape(z_e.shape)
    b = x.shape[0]
    h2 = _tr_phase_conv(z_q.transpose(0, 2, 3, 1), dec_w1, dec_b1, 'relu')
    x_rec_nhwc = _tr_phase_conv(h2, dec_w2, dec_b2, 'sigmoid')
    x_recon = x_rec_nhwc.transpose(0, 3, 1, 2)
    return (x_recon, z_q)
